# baseline (speedup 1.0000x reference)
import functools

import jax
import jax.numpy as jnp
from jax import lax
from jax.experimental import pallas as pl
from jax.experimental.pallas import tpu as pltpu

try:
    from jax.experimental.pallas import tpu_sc as plsc
    _HAS_SC = True
except ImportError:
    _HAS_SC = False

Q = 1024
K_BANK = 100000
D = 512
TOPK = 5

QT = 1024
KB = 2048
K_PAD = ((K_BANK + KB - 1) // KB) * KB
NUM_KB = K_PAD // KB

NEG = float("-inf")
IMAX = 2**31 - 1


def _topk_body(z_ref, bank_ref, vals_ref, idx_ref, znb_ref, ni_ref, s_ref):
    k = pl.program_id(0)

    @pl.when(k == 0)
    def _():
        zt = z_ref[...]
        zn = zt / jnp.maximum(
            jnp.sqrt(jnp.sum(zt * zt, axis=1, keepdims=True)), 1e-12)
        znb_ref[...] = zn.astype(jnp.bfloat16)
        vals_ref[...] = jnp.full((QT, 128), NEG, jnp.float32)
        ni_ref[...] = jnp.full((QT, 128), NEG, jnp.float32)

    bt = bank_ref[...]
    bn = bt / jnp.maximum(jnp.sqrt(jnp.sum(bt * bt, axis=1, keepdims=True)),
                          1e-12)

    sims = lax.dot_general(znb_ref[...], bn.astype(jnp.bfloat16),
                           (((1,), (1,)), ((), ())),
                           preferred_element_type=jnp.float32)

    lcol = lax.broadcasted_iota(jnp.int32, (1, KB), 1)
    ncol = (-k * KB) - lcol.astype(jnp.float32)
    s_ref[...] = jnp.where(lcol + k * KB < K_BANK, sims, NEG)

    lane = lax.broadcasted_iota(jnp.int32, (QT, 128), 1)

    lanef = lane.astype(jnp.float32)
    nchunks = KB // 128
    half = nchunks // 2
    C1, N1, C2, N2 = [], [], [], []
    for h in range(2):
        c0 = h * half
        p0 = s_ref[:, c0 * 128:(c0 + 1) * 128]
        c1v = p0
        n1v = (-(k * KB + c0 * 128)) - lanef
        c2v = jnp.full((QT, 128), NEG, jnp.float32)
        n2v = jnp.zeros((QT, 128), jnp.float32)
        for j in range(1, half):
            c = c0 + j
            p = s_ref[:, c * 128:(c + 1) * 128]
            ncp = (-(k * KB + c * 128)) - lanef
            gt = p > c1v
            gt2 = p > c2v
            x1 = jnp.minimum(p, c1v)
            c2v_new = jnp.maximum(c2v, x1)
            n2v = jnp.where(gt, n1v, jnp.where(gt2, ncp, n2v))
            c2v = c2v_new
            n1v = jnp.where(gt, ncp, n1v)
            c1v = jnp.maximum(p, c1v)
        C1.append(c1v); N1.append(n1v); C2.append(c2v); N2.append(n2v)

    cand_v = vals_ref[...]
    cand_n = ni_ref[...]
    took1 = [jnp.zeros((QT, 128), jnp.bool_) for _ in range(2)]
    trip = jnp.zeros((), jnp.bool_)
    fast_v, fast_n = cand_v, cand_n
    for i in range(TOPK):
        m = jnp.maximum(jnp.max(C1[0], axis=1, keepdims=True),
                        jnp.max(C1[1], axis=1, keepdims=True))
        w = [jnp.where(C1[h] == m, N1[h], NEG) for h in range(2)]
        g = jnp.maximum(jnp.max(w[0], axis=1, keepdims=True),
                        jnp.max(w[1], axis=1, keepdims=True))
        for h in range(2):
            mk = w[h] == g
            if i < TOPK - 1:
                trip = trip | jnp.any(mk & took1[h])
                took1[h] = took1[h] | mk
            C1[h] = jnp.where(mk, C2[h], C1[h])
            N1[h] = jnp.where(mk, N2[h], N1[h])
            C2[h] = jnp.where(mk, NEG, C2[h])
        fast_v = jnp.where(lane == 8 + i, m, fast_v)
        fast_n = jnp.where(lane == 8 + i, g, fast_n)

    def _flat(_):
        cv, cn = cand_v, cand_n
        s = s_ref[...]
        for i in range(TOPK):
            m = jnp.max(s, axis=1, keepdims=True)
            w = jnp.where(s == m, ncol, NEG)
            g = jnp.max(w, axis=1, keepdims=True)
            s = jnp.where(w == g, NEG, s)
            cv = jnp.where(lane == 8 + i, m, cv)
            cn = jnp.where(lane == 8 + i, g, cn)
        return cv, cn

    cand_v, cand_n = lax.cond(trip, _flat, lambda _: (fast_v, fast_n), None)

    new_v = jnp.full((QT, 128), NEG, jnp.float32)
    new_n = jnp.full((QT, 128), NEG, jnp.float32)
    for j in range(TOPK):
        m = jnp.max(cand_v, axis=1, keepdims=True)
        w = jnp.where(cand_v == m, cand_n, NEG)
        g = jnp.max(w, axis=1, keepdims=True)
        cand_v = jnp.where(w == g, NEG, cand_v)
        new_v = jnp.where(lane == j, m, new_v)
        new_n = jnp.where(lane == j, g, new_n)

    vals_ref[...] = new_v
    ni_ref[...] = new_n

    @pl.when(k == NUM_KB - 1)
    def _():
        idx_ref[...] = (-new_n).astype(jnp.int32)


def _cosine_topk(z, bank_padded):
    vals, idx = pl.pallas_call(
        _topk_body,
        grid=(NUM_KB,),
        in_specs=[
            pl.BlockSpec((QT, D), lambda k: (0, 0)),
            pl.BlockSpec((KB, D), lambda k: (k, 0)),
        ],
        out_specs=[
            pl.BlockSpec((QT, 128), lambda k: (0, 0)),
            pl.BlockSpec((QT, 128), lambda k: (0, 0)),
        ],
        out_shape=[
            jax.ShapeDtypeStruct((Q, 128), jnp.float32),
            jax.ShapeDtypeStruct((Q, 128), jnp.int32),
        ],
        scratch_shapes=[
            pltpu.VMEM((QT, D), jnp.bfloat16),
            pltpu.VMEM((QT, 128), jnp.float32),
            pltpu.VMEM((QT, KB), jnp.float32),
        ],
    )(z, bank_padded)
    return vals[:, :TOPK], idx[:, :TOPK]


def _gather_labels_sc(labels_table, idx_flat):
    info = plsc.get_sparse_core_info()
    nc, ns = info.num_cores, info.num_subcores
    nw = nc * ns
    b = idx_flat.shape[0]
    b_per_w = b // nw
    mesh = plsc.VectorSubcoreMesh(core_axis_name="c", subcore_axis_name="s")

    @functools.partial(
        pl.kernel,
        mesh=mesh,
        out_type=jax.ShapeDtypeStruct((b,), jnp.int32),
        scratch_types=[
            pltpu.VMEM((b_per_w,), jnp.int32),
            pltpu.VMEM((b_per_w,), jnp.int32),
            pltpu.SemaphoreType.DMA,
        ],
    )
    def gather_k(idx_hbm, table_hbm, out_hbm, idx_v, rows_v, sem):
        wid = lax.axis_index("s") * nc + lax.axis_index("c")
        base = wid * b_per_w
        pltpu.sync_copy(idx_hbm.at[pl.ds(base, b_per_w)], idx_v)
        pltpu.async_copy(table_hbm.at[idx_v], rows_v, sem).wait()
        pltpu.sync_copy(rows_v, out_hbm.at[pl.ds(base, b_per_w)])

    return gather_k(idx_flat, labels_table)


def kernel(z, lab_bank, lab_labels, topk):
    bank_padded = jnp.pad(lab_bank, ((0, K_PAD - K_BANK), (0, 0)))
    sim, idx = _cosine_topk(z, bank_padded)
    labels = _gather_labels_sc(lab_labels, idx.reshape(-1)).reshape(Q, TOPK)
    idx = idx + (jnp.asarray(topk) * 0).astype(idx.dtype)
    return (idx, sim, labels)


# speedup vs baseline: 1.4717x; 1.4717x over previous
import functools

import jax
import jax.numpy as jnp
from jax import lax
from jax.experimental import pallas as pl
from jax.experimental.pallas import tpu as pltpu

try:
    from jax.experimental.pallas import tpu_sc as plsc
    _HAS_SC = True
except ImportError:
    _HAS_SC = False

Q = 1024
K_BANK = 100000
D = 512
TOPK = 5

QT = 1024
KB = 4096
K_PAD = ((K_BANK + KB - 1) // KB) * KB
NUM_KB = K_PAD // KB

NEG = float("-inf")
IMAX = 2**31 - 1


def _topk_body(z_ref, bank_ref, vals_ref, idx_ref, znb_ref, ni_ref):
    k = pl.program_id(0)

    @pl.when(k == 0)
    def _():
        zt = z_ref[...]
        zn = zt / jnp.maximum(
            jnp.sqrt(jnp.sum(zt * zt, axis=1, keepdims=True)), 1e-12)
        znb_ref[...] = zn.astype(jnp.bfloat16)
        vals_ref[...] = jnp.full((QT, 128), NEG, jnp.float32)
        ni_ref[...] = jnp.full((QT, 128), NEG, jnp.float32)

    bt = bank_ref[...]
    bn = bt / jnp.maximum(jnp.sqrt(jnp.sum(bt * bt, axis=1, keepdims=True)),
                          1e-12)

    sims = lax.dot_general(znb_ref[...], bn.astype(jnp.bfloat16),
                           (((1,), (1,)), ((), ())),
                           preferred_element_type=jnp.float32)

    lcol = lax.broadcasted_iota(jnp.int32, (1, KB), 1)
    ncol = (-k * KB) - lcol.astype(jnp.float32)
    sims = jnp.where(lcol + k * KB < K_BANK, sims, NEG)

    lane = lax.broadcasted_iota(jnp.int32, (QT, 128), 1)

    cand_v = vals_ref[...]
    cand_n = ni_ref[...]
    s = sims
    for i in range(TOPK):
        m = jnp.max(s, axis=1, keepdims=True)
        w = jnp.where(s == m, ncol, NEG)
        g = jnp.max(w, axis=1, keepdims=True)
        s = jnp.where(w == g, NEG, s)
        cand_v = jnp.where(lane == 8 + i, m, cand_v)
        cand_n = jnp.where(lane == 8 + i, g, cand_n)

    new_v = jnp.full((QT, 128), NEG, jnp.float32)
    new_n = jnp.full((QT, 128), NEG, jnp.float32)
    for j in range(TOPK):
        m = jnp.max(cand_v, axis=1, keepdims=True)
        w = jnp.where(cand_v == m, cand_n, NEG)
        g = jnp.max(w, axis=1, keepdims=True)
        cand_v = jnp.where(w == g, NEG, cand_v)
        new_v = jnp.where(lane == j, m, new_v)
        new_n = jnp.where(lane == j, g, new_n)

    vals_ref[...] = new_v
    ni_ref[...] = new_n

    @pl.when(k == NUM_KB - 1)
    def _():
        idx_ref[...] = (-new_n).astype(jnp.int32)


def _cosine_topk(z, bank_padded):
    vals, idx = pl.pallas_call(
        _topk_body,
        grid=(NUM_KB,),
        in_specs=[
            pl.BlockSpec((QT, D), lambda k: (0, 0)),
            pl.BlockSpec((KB, D), lambda k: (k, 0)),
        ],
        out_specs=[
            pl.BlockSpec((QT, 128), lambda k: (0, 0)),
            pl.BlockSpec((QT, 128), lambda k: (0, 0)),
        ],
        out_shape=[
            jax.ShapeDtypeStruct((Q, 128), jnp.float32),
            jax.ShapeDtypeStruct((Q, 128), jnp.int32),
        ],
        scratch_shapes=[
            pltpu.VMEM((QT, D), jnp.bfloat16),
            pltpu.VMEM((QT, 128), jnp.float32),
        ],
    )(z, bank_padded)
    return vals[:, :TOPK], idx[:, :TOPK]


def _gather_labels_sc(labels_table, idx_flat):
    info = plsc.get_sparse_core_info()
    nc, ns = info.num_cores, info.num_subcores
    nw = nc * ns
    b = idx_flat.shape[0]
    b_per_w = b // nw
    mesh = plsc.VectorSubcoreMesh(core_axis_name="c", subcore_axis_name="s")

    @functools.partial(
        pl.kernel,
        mesh=mesh,
        out_type=jax.ShapeDtypeStruct((b,), jnp.int32),
        scratch_types=[
            pltpu.VMEM((b_per_w,), jnp.int32),
            pltpu.VMEM((b_per_w,), jnp.int32),
            pltpu.SemaphoreType.DMA,
        ],
    )
    def gather_k(idx_hbm, table_hbm, out_hbm, idx_v, rows_v, sem):
        wid = lax.axis_index("s") * nc + lax.axis_index("c")
        base = wid * b_per_w
        pltpu.sync_copy(idx_hbm.at[pl.ds(base, b_per_w)], idx_v)
        pltpu.async_copy(table_hbm.at[idx_v], rows_v, sem).wait()
        pltpu.sync_copy(rows_v, out_hbm.at[pl.ds(base, b_per_w)])

    return gather_k(idx_flat, labels_table)


def kernel(z, lab_bank, lab_labels, topk):
    bank_padded = jnp.pad(lab_bank, ((0, K_PAD - K_BANK), (0, 0)))
    sim, idx = _cosine_topk(z, bank_padded)
    labels = _gather_labels_sc(lab_labels, idx.reshape(-1)).reshape(Q, TOPK)
    idx = idx + (jnp.asarray(topk) * 0).astype(idx.dtype)
    return (idx, sim, labels)
